# Initial kernel scaffold; baseline (speedup 1.0000x reference)
import jax
import jax.numpy as jnp
from jax.experimental import pallas as pl


def kernel(cell_feat, net_feat, pin_feat, pins_src, pins_dst, pt_src, pt_dst, W_cell, b_cell, W_net, b_net, W_pin, b_pin, W_gc, b_gc, W_pn, b_pn, W_pe1, b_pe1, W_pe2, b_pe2, W_po, b_po, W_dis, b_dis, W_ang, b_ang):
    raise NotImplementedError("write your pallas kernel here")


# speedup vs baseline: 3.1237x; 3.1237x over previous
import functools

import jax
import jax.numpy as jnp
from jax import lax
from jax.experimental import pallas as pl
from jax.experimental.pallas import tpu as pltpu
from jax.experimental.pallas import tpu_sc as plsc

N_CELLS = 100000
N_NETS = 100000
E_PINS = 1600000
E_PT = 1600000

NSC = 2
NSUB = 16
HALF = N_CELLS // NSC
ZR = 3128
ACC_R = NSUB * ZR
WR = 3128
WR_LAST = HALF - (NSUB - 1) * WR
G = 128
NG_PINS = E_PINS // G
NG_PT = E_PT // G
DUMP = HALF

_f32 = jnp.float32


def _ssp(x):
    return jnp.maximum(x, 0.0) + jnp.log(jnp.exp(-jnp.abs(x)) + 1.0) - 0.6931471805599453


def _he_body(pf, wpin, bpin, wpe1, bpe1, wpe2, bpe2, out):
    hp = jnp.tanh(jnp.dot(pf[...], wpin[...], preferred_element_type=_f32) + bpin[...])
    t = _ssp(jnp.dot(hp, wpe1[...], preferred_element_type=_f32) + bpe1[...])
    out[...] = _ssp(jnp.dot(t, wpe2[...], preferred_element_type=_f32) + bpe2[...])


def _he_call(pin_feat, W_pin, b_pin, W_pe1, b_pe1, W_pe2, b_pe2):
    B = 6400
    grid = (E_PINS // B,)
    full = lambda shp: pl.BlockSpec(shp, lambda i: (0, 0))
    return pl.pallas_call(
        _he_body,
        grid=grid,
        in_specs=[
            pl.BlockSpec((B, 8), lambda i: (i, 0)),
            full((8, 32)), full((1, 32)),
            full((32, 32)), full((1, 32)),
            full((32, 32)), full((1, 32)),
        ],
        out_specs=pl.BlockSpec((B, 32), lambda i: (i, 0)),
        out_shape=jax.ShapeDtypeStruct((E_PINS, 32), _f32),
    )(pin_feat, W_pin, b_pin, W_pe1, b_pe1, W_pe2, b_pe2)


def _hv_body(nf, wnet, bnet, wpn, bpn, out):
    hn = jnp.tanh(jnp.dot(nf[...], wnet[...], preferred_element_type=_f32) + bnet[...])
    out[...] = jnp.dot(hn, wpn[...], preferred_element_type=_f32) + bpn[...]


def _hv_call(net_feat, W_net, b_net, W_pn, b_pn):
    B = 4000
    full = lambda shp: pl.BlockSpec(shp, lambda i: (0, 0))
    return pl.pallas_call(
        _hv_body,
        grid=(N_NETS // B,),
        in_specs=[
            pl.BlockSpec((B, 16), lambda i: (i, 0)),
            full((16, 32)), full((1, 32)),
            full((32, 32)), full((1, 32)),
        ],
        out_specs=pl.BlockSpec((B, 32), lambda i: (i, 0)),
        out_shape=jax.ShapeDtypeStruct((N_NETS, 32), _f32),
    )(net_feat, W_net, b_net, W_pn, b_pn)


def _seg_body(hv, he, src, dst, zrows, out, acc, idxd, idxs, rows, hev, sem):
    c = lax.axis_index("c")
    s = lax.axis_index("s")
    base = c * HALF

    pltpu.sync_copy(zrows, acc.at[pl.ds(s * ZR, ZR)])
    plsc.subcore_barrier()

    ng = jnp.where(s < (NG_PINS - NSUB * (NG_PINS // NSUB)), NG_PINS // NSUB + 1,
                   NG_PINS // NSUB)

    def body(i, _):
        g = s + NSUB * i
        e0 = g * G
        pltpu.sync_copy(dst.at[pl.ds(e0, G)], idxd)
        cp = pltpu.async_copy(hv.at[idxd], rows, sem)
        pltpu.sync_copy(he.at[pl.ds(e0, G)], hev)
        pltpu.sync_copy(src.at[pl.ds(e0, G)], idxs)
        cp.wait()

        def mul_body(r, _):
            rows[r, pl.ds(0, 16)] = rows[r, pl.ds(0, 16)] * hev[r, pl.ds(0, 16)]
            rows[r, pl.ds(16, 16)] = rows[r, pl.ds(16, 16)] * hev[r, pl.ds(16, 16)]
            return _

        lax.fori_loop(0, G, mul_body, None)

        def idx_body(k, _):
            v = idxs[pl.ds(k * 16, 16)]
            inb = (v >= base) & (v < base + HALF)
            idxs[pl.ds(k * 16, 16)] = jnp.where(inb, v - base, DUMP)
            return _

        lax.fori_loop(0, G // 16, idx_body, None)
        pltpu.sync_copy(rows, acc.at[idxs], add=True)
        return _

    lax.fori_loop(0, ng, body, None)
    plsc.subcore_barrier()

    @pl.when(s < NSUB - 1)
    def _():
        pltpu.sync_copy(acc.at[pl.ds(s * WR, WR)], out.at[pl.ds(base + s * WR, WR)])

    @pl.when(s == NSUB - 1)
    def _():
        pltpu.sync_copy(acc.at[pl.ds(s * WR, WR_LAST)],
                        out.at[pl.ds(base + s * WR, WR_LAST)])


def _seg_call(hv, he, src, dst, zrows):
    mesh = plsc.VectorSubcoreMesh(core_axis_name="c", subcore_axis_name="s")
    f = functools.partial(
        pl.kernel,
        mesh=mesh,
        compiler_params=pltpu.CompilerParams(use_tc_tiling_on_sc=False, needs_layout_passes=False),
        out_type=jax.ShapeDtypeStruct((N_CELLS, 32), _f32),
        scratch_types=[
            pltpu.VMEM_SHARED((ACC_R, 32), _f32),
            pltpu.VMEM((G,), jnp.int32),
            pltpu.VMEM((G,), jnp.int32),
            pltpu.VMEM((G, 32), _f32),
            pltpu.VMEM((G, 32), _f32),
            pltpu.SemaphoreType.DMA,
        ],
    )(_seg_body)
    return f(hv, he, src, dst, zrows)


def _ro_body(hc, wpo, bpo, wsrc, wdst, bro, osrc, odst):
    ncell = _ssp(jnp.dot(hc[...], wpo[...], preferred_element_type=_f32) + bpo[...])
    osrc[...] = jnp.dot(ncell, wsrc[...], preferred_element_type=_f32) + bro[...]
    odst[...] = jnp.dot(ncell, wdst[...], preferred_element_type=_f32)


def _ro_mats(hcell, W_po, b_po, Wsrc, Wdst, bro):
    B = 4000
    full = lambda shp: pl.BlockSpec(shp, lambda i: (0, 0))
    return pl.pallas_call(
        _ro_body,
        grid=(N_CELLS // B,),
        in_specs=[
            pl.BlockSpec((B, 32), lambda i: (i, 0)),
            full((32, 32)), full((1, 32)),
            full((32, 16)), full((32, 16)), full((1, 16)),
        ],
        out_specs=[
            pl.BlockSpec((B, 16), lambda i: (i, 0)),
            pl.BlockSpec((B, 16), lambda i: (i, 0)),
        ],
        out_shape=[
            jax.ShapeDtypeStruct((N_CELLS, 16), _f32),
            jax.ShapeDtypeStruct((N_CELLS, 16), _f32),
        ],
    )(hcell, W_po, b_po, Wsrc, Wdst, bro)


def _tanh_sc(x):
    e = jnp.exp(jnp.minimum(2.0 * x, 30.0))
    big = (e - 1.0) / (e + 1.0)
    x2 = x * x
    small = x * (1.0 - x2 * (1.0 / 3.0) + x2 * x2 * (2.0 / 15.0))
    return jnp.where(jnp.abs(x) < 0.25, small, big)


def _edge_body(ssrc, sdst, pts, ptd, dis, ang, idxa, bufa, bufb, disv, angv, sem):
    c = lax.axis_index("c")
    s = lax.axis_index("s")
    w = s * NSC + c
    nw = NSC * NSUB
    ng = jnp.where(w < (NG_PT - nw * (NG_PT // nw)), NG_PT // nw + 1, NG_PT // nw)

    def body(i, _):
        g = w + nw * i
        e0 = g * G
        pltpu.sync_copy(pts.at[pl.ds(e0, G)], idxa)
        pltpu.async_copy(ssrc.at[idxa], bufa, sem).wait()
        pltpu.sync_copy(ptd.at[pl.ds(e0, G)], idxa)
        pltpu.async_copy(sdst.at[idxa], bufb, sem).wait()

        def cb(k, _):
            ridx = lax.iota(jnp.int32, 16) + k * 16
            c0 = jnp.zeros((16,), jnp.int32)
            c1 = jnp.ones((16,), jnp.int32)
            ud = plsc.load_gather(bufa, [ridx, c0]) + plsc.load_gather(bufb, [ridx, c0])
            ua = plsc.load_gather(bufa, [ridx, c1]) + plsc.load_gather(bufb, [ridx, c1])
            disv[pl.ds(k * 16, 16)] = jnp.exp(12.0 * _tanh_sc(ud))
            angv[pl.ds(k * 16, 16)] = 4.0 * _tanh_sc(ua)
            return _

        lax.fori_loop(0, G // 16, cb, None)
        pltpu.sync_copy(disv, dis.at[pl.ds(e0, G)])
        pltpu.sync_copy(angv, ang.at[pl.ds(e0, G)])
        return _

    lax.fori_loop(0, ng, body, None)


def _ro_edges(ssrc, sdst, pt_src, pt_dst):
    mesh = plsc.VectorSubcoreMesh(core_axis_name="c", subcore_axis_name="s")
    f = functools.partial(
        pl.kernel,
        mesh=mesh,
        compiler_params=pltpu.CompilerParams(use_tc_tiling_on_sc=False, needs_layout_passes=False),
        out_type=(
            jax.ShapeDtypeStruct((E_PT,), _f32),
            jax.ShapeDtypeStruct((E_PT,), _f32),
        ),
        scratch_types=[
            pltpu.VMEM((G,), jnp.int32),
            pltpu.VMEM((G, 16), _f32),
            pltpu.VMEM((G, 16), _f32),
            pltpu.VMEM((G,), _f32),
            pltpu.VMEM((G,), _f32),
            pltpu.SemaphoreType.DMA,
        ],
    )(_edge_body)
    return f(ssrc, sdst, pt_src, pt_dst)


def kernel(cell_feat, net_feat, pin_feat, pins_src, pins_dst, pt_src, pt_dst,
           W_cell, b_cell, W_net, b_net, W_pin, b_pin, W_gc, b_gc,
           W_pn, b_pn, W_pe1, b_pe1, W_pe2, b_pe2, W_po, b_po,
           W_dis, b_dis, W_ang, b_ang):
    he = _he_call(pin_feat, W_pin, b_pin.reshape(1, 32),
                  W_pe1, b_pe1.reshape(1, 32), W_pe2, b_pe2.reshape(1, 32))
    hv = _hv_call(net_feat, W_net, b_net.reshape(1, 32), W_pn, b_pn.reshape(1, 32))
    zrows = jnp.zeros((ZR, 32), _f32)
    hcell = _seg_call(hv, he, pins_src, pins_dst, zrows)

    pad = jnp.zeros((32, 14), _f32)
    Wsrc = jnp.concatenate([W_dis[:32], W_ang[:32], pad], axis=1)
    Wdst = jnp.concatenate([W_dis[32:], W_ang[32:], pad], axis=1)
    bro = jnp.concatenate([b_dis, b_ang, jnp.zeros((14,), _f32)]).reshape(1, 16)
    ssrc, sdst = _ro_mats(hcell, W_po, b_po.reshape(1, 32), Wsrc, Wdst, bro)

    dis, ang = _ro_edges(ssrc, sdst, pt_src, pt_dst)
    return dis, ang
